# Initial kernel scaffold; baseline (speedup 1.0000x reference)
import jax
import jax.numpy as jnp
from jax import lax
from jax.experimental import pallas as pl
from jax.experimental.pallas import tpu as pltpu
from jax.experimental.pallas import tpu_sc as plsc

NC = 2
NS = 16
NW = NC * NS

D = 32
B = 16384 * 50
B_PER_W = B // NW
CHUNK = 1280
N_CHUNKS = B_PER_W // CHUNK


def _gather_body(idx_hbm, table_hbm, out_hbm, idx_v, rows_v, sem):
    wid = lax.axis_index("s") * NC + lax.axis_index("c")
    base = wid * B_PER_W
    pltpu.sync_copy(idx_hbm.at[pl.ds(base, B_PER_W)], idx_v)

    def chunk(c, carry):
        off = c * CHUNK
        pltpu.async_copy(
            table_hbm.at[idx_v.at[pl.ds(off, CHUNK)]], rows_v, sem
        ).wait()
        pltpu.sync_copy(rows_v, out_hbm.at[pl.ds(base + off, CHUNK)])
        return carry

    lax.fori_loop(0, N_CHUNKS, chunk, 0)


@jax.jit
def kernel(inputs, weight):
    flat_idx = inputs.reshape(-1).astype(jnp.int32)
    mesh = plsc.VectorSubcoreMesh(core_axis_name="c", subcore_axis_name="s")
    call = pl.kernel(
        _gather_body,
        out_type=jax.ShapeDtypeStruct((B, D), jnp.float32),
        mesh=mesh,
        scratch_types=[
            pltpu.VMEM((B_PER_W,), jnp.int32),
            pltpu.VMEM((CHUNK, D), jnp.float32),
            pltpu.SemaphoreType.DMA,
        ],
    )
    out = call(flat_idx, weight)
    return out.reshape(inputs.shape + (D,))


# speedup vs baseline: 1.1041x; 1.1041x over previous
import jax
import jax.numpy as jnp
from jax import lax
from jax.experimental import pallas as pl
from jax.experimental.pallas import tpu as pltpu
from jax.experimental.pallas import tpu_sc as plsc

NC = 2
NS = 16
NW = NC * NS

D = 32
B = 16384 * 50
B_PER_W = B // NW
CHUNK = 1280
N_CHUNKS = B_PER_W // CHUNK


def _gather_body(idx_hbm, table_hbm, out_hbm, idx_v, rows_v, sem):
    wid = lax.axis_index("s") * NC + lax.axis_index("c")
    base = wid * B_PER_W
    pltpu.sync_copy(idx_hbm.at[pl.ds(base, B_PER_W)], idx_v)

    def chunk(c, carry):
        off = c * CHUNK
        pltpu.async_copy(
            table_hbm.at[idx_v.at[pl.ds(off, CHUNK)]], rows_v, sem
        ).wait()
        pltpu.sync_copy(rows_v, out_hbm.at[pl.ds(base + off, CHUNK)])
        return carry

    lax.fori_loop(0, N_CHUNKS, chunk, 0)


@jax.jit
def kernel(inputs, weight):
    flat_idx = inputs.reshape(-1).astype(jnp.int32)
    mesh = plsc.VectorSubcoreMesh(core_axis_name="c", subcore_axis_name="s")
    call = pl.kernel(
        _gather_body,
        out_type=jax.ShapeDtypeStruct((B, D), jnp.float32),
        mesh=mesh,
        scratch_types=[
            pltpu.VMEM((B_PER_W,), jnp.int32),
            pltpu.VMEM((CHUNK, D), jnp.float32),
            pltpu.SemaphoreType.DMA,
        ],
        compiler_params=pltpu.CompilerParams(use_tc_tiling_on_sc=False),
    )
    out = call(flat_idx, weight)
    return out.reshape(inputs.shape + (D,))


# speedup vs baseline: 1.1117x; 1.0069x over previous
import jax
import jax.numpy as jnp
from jax import lax
from jax.experimental import pallas as pl
from jax.experimental.pallas import tpu as pltpu
from jax.experimental.pallas import tpu_sc as plsc

NC = 2
NS = 16
NW = NC * NS

D = 32
B = 16384 * 50
B_PER_W = B // NW
CHUNK = 1280
N_CHUNKS = B_PER_W // CHUNK


def _gather_body(idx_hbm, table_hbm, out_hbm, idx_v, rows0, rows1,
                 sg0, sg1, so0, so1):
    wid = lax.axis_index("s") * NC + lax.axis_index("c")
    base = wid * B_PER_W
    pltpu.sync_copy(idx_hbm.at[pl.ds(base, B_PER_W)], idx_v)

    bufs = (rows0, rows1)
    sgs = (sg0, sg1)
    sos = (so0, so1)
    gath = [None, None]
    outc = [None, None]

    gath[0] = pltpu.async_copy(
        table_hbm.at[idx_v.at[pl.ds(0, CHUNK)]], bufs[0], sgs[0])
    for c in range(N_CHUNKS):
        b = c % 2
        nb = (c + 1) % 2
        if c + 1 < N_CHUNKS:
            if outc[nb] is not None:
                outc[nb].wait()
            gath[nb] = pltpu.async_copy(
                table_hbm.at[idx_v.at[pl.ds((c + 1) * CHUNK, CHUNK)]],
                bufs[nb], sgs[nb])
        gath[b].wait()
        outc[b] = pltpu.async_copy(
            bufs[b], out_hbm.at[pl.ds(base + c * CHUNK, CHUNK)], sos[b])
    outc[0].wait()
    outc[1].wait()


@jax.jit
def kernel(inputs, weight):
    flat_idx = inputs.reshape(-1).astype(jnp.int32)
    mesh = plsc.VectorSubcoreMesh(core_axis_name="c", subcore_axis_name="s")
    call = pl.kernel(
        _gather_body,
        out_type=jax.ShapeDtypeStruct((B, D), jnp.float32),
        mesh=mesh,
        scratch_types=[
            pltpu.VMEM((B_PER_W,), jnp.int32),
            pltpu.VMEM((CHUNK, D), jnp.float32),
            pltpu.VMEM((CHUNK, D), jnp.float32),
            pltpu.SemaphoreType.DMA,
            pltpu.SemaphoreType.DMA,
            pltpu.SemaphoreType.DMA,
            pltpu.SemaphoreType.DMA,
        ],
        compiler_params=pltpu.CompilerParams(use_tc_tiling_on_sc=False),
    )
    out = call(flat_idx, weight)
    return out.reshape(inputs.shape + (D,))


# speedup vs baseline: 1.1125x; 1.0007x over previous
import jax
import jax.numpy as jnp
from jax import lax
from jax.experimental import pallas as pl
from jax.experimental.pallas import tpu as pltpu
from jax.experimental.pallas import tpu_sc as plsc
from jax.experimental import layout as jax_layout

NC = 2
NS = 16
NW = NC * NS

D = 32
B = 16384 * 50
B_PER_W = B // NW
CHUNK = 1280
N_CHUNKS = B_PER_W // CHUNK


def _gather_body(idx_hbm, table_hbm, out_hbm, idx_v, rows0, rows1,
                 sg0, sg1, so0, so1):
    wid = lax.axis_index("s") * NC + lax.axis_index("c")
    base = wid * B_PER_W
    pltpu.sync_copy(idx_hbm.at[pl.ds(base, B_PER_W)], idx_v)

    bufs = (rows0, rows1)
    sgs = (sg0, sg1)
    sos = (so0, so1)
    gath = [None, None]
    outc = [None, None]

    gath[0] = pltpu.async_copy(
        table_hbm.at[idx_v.at[pl.ds(0, CHUNK)]], bufs[0], sgs[0])
    for c in range(N_CHUNKS):
        b = c % 2
        nb = (c + 1) % 2
        if c + 1 < N_CHUNKS:
            if outc[nb] is not None:
                outc[nb].wait()
            gath[nb] = pltpu.async_copy(
                table_hbm.at[idx_v.at[pl.ds((c + 1) * CHUNK, CHUNK)]],
                bufs[nb], sgs[nb])
        gath[b].wait()
        outc[b] = pltpu.async_copy(
            bufs[b], out_hbm.at[pl.ds(base + c * CHUNK, CHUNK)], sos[b])
    outc[0].wait()
    outc[1].wait()


def _kernel_impl(inputs, weight):
    flat_idx = inputs.reshape(-1).astype(jnp.int32)
    mesh = plsc.VectorSubcoreMesh(core_axis_name="c", subcore_axis_name="s")
    call = pl.kernel(
        _gather_body,
        out_type=jax.ShapeDtypeStruct((B, D), jnp.float32),
        mesh=mesh,
        scratch_types=[
            pltpu.VMEM((B_PER_W,), jnp.int32),
            pltpu.VMEM((CHUNK, D), jnp.float32),
            pltpu.VMEM((CHUNK, D), jnp.float32),
            pltpu.SemaphoreType.DMA,
            pltpu.SemaphoreType.DMA,
            pltpu.SemaphoreType.DMA,
            pltpu.SemaphoreType.DMA,
        ],
        compiler_params=pltpu.CompilerParams(use_tc_tiling_on_sc=False),
    )
    out = call(flat_idx, weight)
    return out.reshape(inputs.shape + (D,))


_jitted = None


def kernel(inputs, weight):
    global _jitted
    if _jitted is None:
        try:
            dev = jax.devices()[0]
            jitted = jax.jit(
                _kernel_impl,
                out_shardings=jax_layout.Format(
                    jax_layout.Layout(major_to_minor=(0, 1, 2), tiling=()),
                    jax.sharding.SingleDeviceSharding(dev),
                ),
            )
            out = jitted(inputs, weight)
            _jitted = jitted
            return out
        except Exception:
            _jitted = jax.jit(_kernel_impl)
    return _jitted(inputs, weight)


# speedup vs baseline: 1.5122x; 1.3593x over previous
import jax
import jax.numpy as jnp
from jax import lax
from jax.experimental import pallas as pl
from jax.experimental.pallas import tpu as pltpu
from jax.experimental.pallas import tpu_sc as plsc

NC = 2
NS = 16
NW = NC * NS

D = 32
NB = 16384
NT = 50
B = NB * NT

BT = 128
NJ = NB // BT
JB = NJ // NW
TC = 5
NTC = NT // TC
CH = TC * BT


def _gather_body(idx_hbm, table_hbm, out_hbm, idx_v, idxp_v, rows_v,
                 stage_v, sem_g, sem_o):
    wid = lax.axis_index("s") * NC + lax.axis_index("c")
    iota = lax.iota(jnp.int32, 16)
    iota50 = iota * NT

    def jblock(k, carry):
        j = wid * JB + k
        pltpu.sync_copy(idx_hbm.at[pl.ds(j * BT * NT, BT * NT)], idx_v)

        def chunk(m, carry2):
            t0 = m * TC
            for tt in range(TC):
                for g in range(8):
                    addr = iota50 + (g * 16 * NT + t0 + tt)
                    v = plsc.load_gather(idx_v, [addr])
                    idxp_v[pl.ds(tt * BT + g * 16, 16)] = v
            pltpu.async_copy(table_hbm.at[idxp_v], rows_v, sem_g).wait()

            def trans(q, carry3):
                tt = q // D
                c = q % D
                for g in range(8):
                    rid = iota + (tt * BT + g * 16)
                    cid = jnp.broadcast_to(c, (16,)).astype(jnp.int32)
                    v = plsc.load_gather(rows_v, [rid, cid])
                    stage_v[tt * D + c, pl.ds(g * 16, 16)] = v
                return carry3

            lax.fori_loop(0, TC * D, trans, 0)

            cps = []
            for tt in range(TC):
                for i in range(4):
                    cps.append(pltpu.async_copy(
                        stage_v.at[pl.ds(tt * D + 8 * i, 8)],
                        out_hbm.at[t0 + tt, i, j],
                        sem_o))
            for cp in cps:
                cp.wait()
            return carry2

        lax.fori_loop(0, NTC, chunk, 0)
        return carry

    lax.fori_loop(0, JB, jblock, 0)


def kernel(inputs, weight):
    flat_idx = inputs.reshape(-1).astype(jnp.int32)
    mesh = plsc.VectorSubcoreMesh(core_axis_name="c", subcore_axis_name="s")
    call = pl.kernel(
        _gather_body,
        out_type=jax.ShapeDtypeStruct((NT, D // 8, NJ, 8, BT), jnp.float32),
        mesh=mesh,
        scratch_types=[
            pltpu.VMEM((BT * NT,), jnp.int32),
            pltpu.VMEM((CH,), jnp.int32),
            pltpu.VMEM((CH, D), jnp.float32),
            pltpu.VMEM((TC * D, BT), jnp.float32),
            pltpu.SemaphoreType.DMA,
            pltpu.SemaphoreType.DMA,
        ],
        compiler_params=pltpu.CompilerParams(
            use_tc_tiling_on_sc=False, needs_layout_passes=False),
    )
    out5 = call(flat_idx, weight)
    return out5.transpose(2, 4, 0, 1, 3).reshape(NB, NT, D)


# speedup vs baseline: 1.5745x; 1.0412x over previous
import jax
import jax.numpy as jnp
from jax import lax
from jax.experimental import pallas as pl
from jax.experimental.pallas import tpu as pltpu
from jax.experimental.pallas import tpu_sc as plsc

NC = 2
NS = 16
NW = NC * NS

D = 32
NB = 16384
NT = 50
B = NB * NT

BT = 128
NJ = NB // BT
JB = NJ // NW
TC = 5
NTC = NT // TC
CH = TC * BT


def _gather_body(idx_hbm, table_hbm, out_hbm, idx_v,
                 idxp0, idxp1, rows0, rows1, stage0, stage1,
                 sg0, sg1, so0, so1):
    wid = lax.axis_index("s") * NC + lax.axis_index("c")
    iota = lax.iota(jnp.int32, 16)
    iota50 = iota * NT

    idxps = (idxp0, idxp1)
    rows = (rows0, rows1)
    stages = (stage0, stage1)
    sgs = (sg0, sg1)
    sos = (so0, so1)

    def jblock(k, carry):
        j = wid * JB + k
        pltpu.sync_copy(idx_hbm.at[pl.ds(j * BT * NT, BT * NT)], idx_v)

        def permute(m, idxp):
            t0 = m * TC
            for tt in range(TC):
                for g in range(8):
                    v = plsc.load_gather(
                        idx_v, [iota50 + (g * 16 * NT + t0 + tt)])
                    idxp[pl.ds(tt * BT + g * 16, 16)] = v

        def gather_start(p):
            pltpu.async_copy(table_hbm.at[idxps[p]], rows[p], sgs[p])

        def gather_wait(p):
            pltpu.make_async_copy(
                table_hbm.at[idxps[p]], rows[p], sgs[p]).wait()

        def transpose(p):
            r = rows[p]
            st = stages[p]
            for tt in range(TC):
                for i in range(4):
                    for s in range(8):
                        cid = jnp.full((16,), 8 * i + s, jnp.int32)
                        for g in range(8):
                            rid = iota + (tt * BT + g * 16)
                            v = plsc.load_gather(r, [rid, cid])
                            st[tt, i, s, pl.ds(g * 16, 16)] = v

        def out_start(p, m):
            t0 = m * TC
            for tt in range(TC):
                pltpu.async_copy(
                    stages[p].at[tt], out_hbm.at[t0 + tt, :, j], sos[p])

        def out_wait(p, m):
            t0 = m * TC
            for tt in range(TC):
                pltpu.make_async_copy(
                    stages[p].at[tt], out_hbm.at[t0 + tt, :, j],
                    sos[p]).wait()

        def chunk_body(m, p):
            @pl.when(m >= 2)
            def _():
                out_wait(p, m - 2)
            gather_wait(p)

            @pl.when(m + 1 < NTC)
            def _():
                permute(m + 1, idxps[1 - p])
                gather_start(1 - p)
            transpose(p)
            out_start(p, m)

        permute(0, idxp0)
        gather_start(0)

        def pair(m2, carry2):
            chunk_body(m2 * 2, 0)
            chunk_body(m2 * 2 + 1, 1)
            return carry2

        lax.fori_loop(0, NTC // 2, pair, 0)
        out_wait(0, NTC - 2)
        out_wait(1, NTC - 1)
        return carry

    lax.fori_loop(0, JB, jblock, 0)


def kernel(inputs, weight):
    flat_idx = inputs.reshape(-1).astype(jnp.int32)
    mesh = plsc.VectorSubcoreMesh(core_axis_name="c", subcore_axis_name="s")
    call = pl.kernel(
        _gather_body,
        out_type=jax.ShapeDtypeStruct((NT, D // 8, NJ, 8, BT), jnp.float32),
        mesh=mesh,
        scratch_types=[
            pltpu.VMEM((BT * NT,), jnp.int32),
            pltpu.VMEM((CH,), jnp.int32),
            pltpu.VMEM((CH,), jnp.int32),
            pltpu.VMEM((CH, D), jnp.float32),
            pltpu.VMEM((CH, D), jnp.float32),
            pltpu.VMEM((TC, D // 8, 8, BT), jnp.float32),
            pltpu.VMEM((TC, D // 8, 8, BT), jnp.float32),
            pltpu.SemaphoreType.DMA,
            pltpu.SemaphoreType.DMA,
            pltpu.SemaphoreType.DMA,
            pltpu.SemaphoreType.DMA,
        ],
        compiler_params=pltpu.CompilerParams(
            use_tc_tiling_on_sc=False, needs_layout_passes=False),
    )
    out5 = call(flat_idx, weight)
    return out5.transpose(2, 4, 0, 1, 3).reshape(NB, NT, D)


# speedup vs baseline: 2.5371x; 1.6113x over previous
import jax
import jax.numpy as jnp
from jax import lax
from jax.experimental import pallas as pl
from jax.experimental.pallas import tpu as pltpu
from jax.experimental.pallas import tpu_sc as plsc

NC = 2
NS = 16
NW = NC * NS

D = 32
NB = 16384
NT = 50
B = NB * NT

BT = 128
NJ = NB // BT
JB = NJ // NW
TC = 5
NTC = NT // TC
CH = TC * BT


def _gather_body(idx_hbm, table_hbm, out_hbm, idx_v,
                 idxp0, idxp1, rows0, rows1, stage0, stage1,
                 sg0, sg1, so0, so1):
    wid = lax.axis_index("s") * NC + lax.axis_index("c")
    iota = lax.iota(jnp.int32, 16)
    iota50 = iota * NT

    idxps = (idxp0, idxp1)
    rows = (rows0, rows1)
    stages = (stage0, stage1)
    sgs = (sg0, sg1)
    sos = (so0, so1)

    def jblock(k, carry):
        j = wid * JB + k
        pltpu.sync_copy(idx_hbm.at[pl.ds(j * BT * NT, BT * NT)], idx_v)

        def permute(m, idxp):
            t0 = m * TC
            for tt in range(TC):
                for g in range(8):
                    v = plsc.load_gather(
                        idx_v, [iota50 + (g * 16 * NT + t0 + tt)])
                    idxp[pl.ds(tt * BT + g * 16, 16)] = v

        def gather_start(p):
            pltpu.async_copy(table_hbm.at[idxps[p]], rows[p], sgs[p])

        def gather_wait(p):
            pltpu.make_async_copy(
                table_hbm.at[idxps[p]], rows[p], sgs[p]).wait()

        def transpose(p):
            r = rows[p]
            st = stages[p]

            def trans(rb, carry3):
                for u in range(8):
                    rr = rb * 8 + u
                    tt = rr // BT
                    bb = rr % BT
                    lvec = jnp.broadcast_to(bb, (16,)).astype(jnp.int32)
                    for c0 in (0, 16):
                        v = r[rr, pl.ds(c0, 16)]
                        r2 = iota + (tt * D + c0)
                        plsc.store_scatter(st, [r2, lvec], v)
                return carry3

            lax.fori_loop(0, CH // 8, trans, 0)

        def out_start(p, m):
            t0 = m * TC
            for tt in range(TC):
                for i in range(D // 8):
                    pltpu.async_copy(
                        stages[p].at[pl.ds(tt * D + 8 * i, 8), pl.ds(0, BT)],
                        out_hbm.at[t0 + tt, i, j], sos[p])

        def out_wait(p, m):
            t0 = m * TC
            for tt in range(TC):
                for i in range(D // 8):
                    pltpu.make_async_copy(
                        stages[p].at[pl.ds(tt * D + 8 * i, 8), pl.ds(0, BT)],
                        out_hbm.at[t0 + tt, i, j], sos[p]).wait()

        def chunk_body(m, p):
            @pl.when(m >= 2)
            def _():
                out_wait(p, m - 2)
            gather_wait(p)

            @pl.when(m + 1 < NTC)
            def _():
                permute(m + 1, idxps[1 - p])
                gather_start(1 - p)
            transpose(p)
            out_start(p, m)

        permute(0, idxp0)
        gather_start(0)

        def pair(m2, carry2):
            chunk_body(m2 * 2, 0)
            chunk_body(m2 * 2 + 1, 1)
            return carry2

        lax.fori_loop(0, NTC // 2, pair, 0)
        out_wait(0, NTC - 2)
        out_wait(1, NTC - 1)
        return carry

    lax.fori_loop(0, JB, jblock, 0)


def kernel(inputs, weight):
    flat_idx = inputs.reshape(-1).astype(jnp.int32)
    mesh = plsc.VectorSubcoreMesh(core_axis_name="c", subcore_axis_name="s")
    call = pl.kernel(
        _gather_body,
        out_type=jax.ShapeDtypeStruct((NT, D // 8, NJ, 8, BT), jnp.float32),
        mesh=mesh,
        scratch_types=[
            pltpu.VMEM((BT * NT,), jnp.int32),
            pltpu.VMEM((CH,), jnp.int32),
            pltpu.VMEM((CH,), jnp.int32),
            pltpu.VMEM((CH, D), jnp.float32),
            pltpu.VMEM((CH, D), jnp.float32),
            pltpu.VMEM((TC * D, BT + 1), jnp.float32),
            pltpu.VMEM((TC * D, BT + 1), jnp.float32),
            pltpu.SemaphoreType.DMA,
            pltpu.SemaphoreType.DMA,
            pltpu.SemaphoreType.DMA,
            pltpu.SemaphoreType.DMA,
        ],
        compiler_params=pltpu.CompilerParams(
            use_tc_tiling_on_sc=False, needs_layout_passes=False),
    )
    out5 = call(flat_idx, weight)
    return out5.transpose(2, 4, 0, 1, 3).reshape(NB, NT, D)
